# baseline (speedup 1.0000x reference)
import dataclasses

import jax
import jax.numpy as jnp
from jax import lax
from jax.experimental import pallas as pl
from jax.experimental.pallas import tpu as pltpu
from jax.experimental.pallas import tpu_sc as plsc

N = 1000000
D = 8
K = 1024
LANES = 16

CHUNK_ROWS = 800
GROUPS = CHUNK_ROWS // LANES
NUM_CHUNKS = N // CHUNK_ROWS
ZW = D + 1
MPAD = D + 1
DPAD = D * D + 1

LVL_BITS = (32, 16, 8, 4, 2, 1)
LVL_OFF = {}
_off = 0
for _b in LVL_BITS:
    LVL_OFF[_b] = _off
    _off += K // (2 * _b)
LVL_WORDS = _off


def _sc_body(zt_hbm, means_hbm, devs_hbm, part_hbm, out_hbm,
             part_v, means_v, devs_v, lvl_v, sem):
    c1 = pltpu.async_copy(part_hbm, part_v, sem)
    c2 = pltpu.async_copy(means_hbm, means_v, sem)
    c3 = pltpu.async_copy(devs_hbm, devs_v, sem)
    c1.wait()
    c2.wait()
    c3.wait()

    iota = lax.iota(jnp.int32, LANES)

    cvec = plsc.load_gather(part_v, [iota * 64 + 63])

    for b in LVL_BITS:
        n_ent = K // (2 * b)
        for j0 in range(0, n_ent, LANES):
            ent = plsc.load_gather(part_v, [(j0 + iota) * (2 * b) + (b - 1)])
            lvl_v[pl.ds(LVL_OFF[b] + j0, LANES)] = ent

    def chunk_body(z_v, y_v):
        @plsc.parallel_loop(0, GROUPS, unroll=4)
        def _(g):
            r0 = g * LANES
            u = z_v[pl.ds(r0, LANES)]
            pos_r = jnp.zeros((LANES,), jnp.int32)
            for b in (8, 4, 2, 1):
                val = cvec.at[pos_r + (b - 1)].get(mode="promise_in_bounds")
                pos_r = jnp.where(val <= u, pos_r + b, pos_r)
            pos = pos_r * 64
            for b in LVL_BITS:
                j = pos // (2 * b)
                val = plsc.load_gather(lvl_v, [j + LVL_OFF[b]])
                pos = jnp.where(val <= u, pos + b, pos)
            idx = jnp.minimum(pos, K - 1)
            xs = [z_v[pl.ds((1 + j) * CHUNK_ROWS + r0, LANES)]
                  for j in range(D)]
            base_m = idx * MPAD
            base_d = idx * DPAD
            accs = [plsc.load_gather(means_v, [base_m + i]) for i in range(D)]
            for i in range(D):
                acc = accs[i]
                for j in range(D):
                    m = plsc.load_gather(devs_v, [base_d + (i * D + j)])
                    acc = acc + m * xs[j]
                accs[i] = acc
            for i in range(D):
                y_v[pl.ds(i * CHUNK_ROWS + r0, LANES)] = accs[i]

    pltpu.emit_pipeline(
        chunk_body,
        grid=(NUM_CHUNKS,),
        in_specs=[pl.BlockSpec((ZW * CHUNK_ROWS,), lambda i: (i,))],
        out_specs=[pl.BlockSpec((D * CHUNK_ROWS,), lambda i: (i,))],
        core_axis_name=("c", "s"),
        dimension_semantics=(pltpu.PARALLEL,),
    )(zt_hbm, out_hbm)


@jax.jit
def kernel(z, means, devs, mix_partition):
    mesh = plsc.VectorSubcoreMesh(core_axis_name="c", subcore_axis_name="s")
    cp = pltpu.CompilerParams()
    if "needs_layout_passes" in pltpu.CompilerParams.__dataclass_fields__:
        cp = dataclasses.replace(cp, needs_layout_passes=False)
    run = pl.kernel(
        _sc_body,
        out_type=jax.ShapeDtypeStruct((N * D,), jnp.float32),
        mesh=mesh,
        scratch_types=[
            pltpu.VMEM((K,), jnp.float32),
            pltpu.VMEM((K * MPAD,), jnp.float32),
            pltpu.VMEM((K * DPAD,), jnp.float32),
            pltpu.VMEM((LVL_WORDS,), jnp.float32),
            pltpu.SemaphoreType.DMA,
        ],
        compiler_params=cp,
    )
    means_p = jnp.pad(means.reshape(K, D), ((0, 0), (0, MPAD - D))).reshape(K * MPAD)
    devs_p = jnp.pad(devs.reshape(K, D * D), ((0, 0), (0, DPAD - D * D))).reshape(K * DPAD)
    zt = (z.T.reshape(ZW, NUM_CHUNKS, CHUNK_ROWS)
          .transpose(1, 0, 2).reshape(N * ZW))
    out = run(zt, means_p, devs_p, mix_partition)
    return (out.reshape(NUM_CHUNKS, D, CHUNK_ROWS)
            .transpose(1, 0, 2).reshape(D, N).T)


# speedup vs baseline: 1.8663x; 1.8663x over previous
import dataclasses

import jax
import jax.numpy as jnp
from jax import lax
from jax.experimental import pallas as pl
from jax.experimental.pallas import tpu as pltpu
from jax.experimental.pallas import tpu_sc as plsc

N = 1000000
D = 8
K = 1024
LANES = 16

CHUNK_ROWS = 160
GROUPS = CHUNK_ROWS // LANES
NUM_CHUNKS = N // CHUNK_ROWS
ZW = D + 1
MPAD = D + 1
DPAD = D * D + 1

LVL_BITS = (32, 16, 8, 4, 2, 1)
LVL_OFF = {}
_off = 0
for _b in LVL_BITS:
    LVL_OFF[_b] = _off
    _off += K // (2 * _b)
LVL_WORDS = _off


def _sc_body(z_hbm, means_hbm, devs_hbm, part_hbm, out_hbm,
             part_v, means_v, devs_v, lvl_v, sem):
    c1 = pltpu.async_copy(part_hbm, part_v, sem)
    c2 = pltpu.async_copy(means_hbm, means_v, sem)
    c3 = pltpu.async_copy(devs_hbm, devs_v, sem)
    c1.wait()
    c2.wait()
    c3.wait()

    iota = lax.iota(jnp.int32, LANES)
    iota_z = iota * ZW
    iota_y = iota * D

    cvec = plsc.load_gather(part_v, [iota * 64 + 63])

    for b in LVL_BITS:
        n_ent = K // (2 * b)
        for j0 in range(0, n_ent, LANES):
            ent = plsc.load_gather(part_v, [(j0 + iota) * (2 * b) + (b - 1)])
            lvl_v[pl.ds(LVL_OFF[b] + j0, LANES)] = ent

    def chunk_body(z_v, out_v):
        @plsc.parallel_loop(0, GROUPS, unroll=4)
        def _(g):
            rows = g * LANES + iota
            rowz = g * (LANES * ZW) + iota_z
            u = plsc.load_gather(z_v, [rowz])
            pos_r = jnp.zeros((LANES,), jnp.int32)
            for b in (8, 4, 2, 1):
                val = cvec.at[pos_r + (b - 1)].get(mode="promise_in_bounds")
                pos_r = jnp.where(val <= u, pos_r + b, pos_r)
            pos = pos_r * 64
            for b in LVL_BITS:
                j = pos // (2 * b)
                val = plsc.load_gather(lvl_v, [j + LVL_OFF[b]])
                pos = jnp.where(val <= u, pos + b, pos)
            idx = jnp.minimum(pos, K - 1)
            xs = [plsc.load_gather(z_v, [rowz + (1 + j)]) for j in range(D)]
            base_m = idx * MPAD
            base_d = idx * DPAD
            accs = [plsc.load_gather(means_v, [base_m + i]) for i in range(D)]
            for i in range(D):
                acc = accs[i]
                for j in range(D):
                    m = plsc.load_gather(devs_v, [base_d + (i * D + j)])
                    acc = acc + m * xs[j]
                accs[i] = acc
            for i in range(D):
                plsc.store_scatter(
                    out_v, [rows, jnp.full((LANES,), i, jnp.int32)], accs[i])

    pltpu.emit_pipeline(
        chunk_body,
        grid=(NUM_CHUNKS,),
        in_specs=[pl.BlockSpec((CHUNK_ROWS * ZW,), lambda i: (i,))],
        out_specs=[pl.BlockSpec((CHUNK_ROWS, D), lambda i: (i, 0))],
        core_axis_name=("c", "s"),
        dimension_semantics=(pltpu.PARALLEL,),
    )(z_hbm, out_hbm)


@jax.jit
def kernel(z, means, devs, mix_partition):
    mesh = plsc.VectorSubcoreMesh(core_axis_name="c", subcore_axis_name="s")
    cp = pltpu.CompilerParams()
    if "needs_layout_passes" in pltpu.CompilerParams.__dataclass_fields__:
        cp = dataclasses.replace(cp, needs_layout_passes=False)
    run = pl.kernel(
        _sc_body,
        out_type=jax.ShapeDtypeStruct((N, D), jnp.float32),
        mesh=mesh,
        scratch_types=[
            pltpu.VMEM((K,), jnp.float32),
            pltpu.VMEM((K * MPAD,), jnp.float32),
            pltpu.VMEM((K * DPAD,), jnp.float32),
            pltpu.VMEM((LVL_WORDS,), jnp.float32),
            pltpu.SemaphoreType.DMA,
        ],
        compiler_params=cp,
    )
    means_p = jnp.pad(means.reshape(K, D), ((0, 0), (0, MPAD - D))).reshape(K * MPAD)
    devs_p = jnp.pad(devs.reshape(K, D * D), ((0, 0), (0, DPAD - D * D))).reshape(K * DPAD)
    return run(z.reshape(N * ZW), means_p, devs_p, mix_partition)


# speedup vs baseline: 2.4563x; 1.3162x over previous
import dataclasses

import jax
import jax.numpy as jnp
from jax import lax
from jax.experimental import pallas as pl
from jax.experimental.pallas import tpu as pltpu
from jax.experimental.pallas import tpu_sc as plsc

N = 1000000
NP2 = 1048576
D = 8
K = 1024
LANES = 16

CHUNK_ROWS = 128
GROUPS = CHUNK_ROWS // LANES
NUM_CHUNKS = NP2 // CHUNK_ROWS
ZW = D + 1
MPAD = D + 1
DPAD = D * D + 1

LVL_BITS = (32, 16, 8, 4, 2, 1)
LVL_OFF = {}
_off = 0
for _b in LVL_BITS:
    LVL_OFF[_b] = _off
    _off += K // (2 * _b)
LVL_WORDS = _off


def _sc_body(z_hbm, means_hbm, devs_hbm, part_hbm, out_hbm,
             part_v, means_v, devs_v, lvl_v, sem):
    c1 = pltpu.async_copy(part_hbm, part_v, sem)
    c2 = pltpu.async_copy(means_hbm, means_v, sem)
    c3 = pltpu.async_copy(devs_hbm, devs_v, sem)
    c1.wait()
    c2.wait()
    c3.wait()

    iota = lax.iota(jnp.int32, LANES)
    iota_z = iota * ZW
    iota_y = iota * D

    cvec = plsc.load_gather(part_v, [iota * 64 + 63])

    for b in LVL_BITS:
        n_ent = K // (2 * b)
        for j0 in range(0, n_ent, LANES):
            ent = plsc.load_gather(part_v, [(j0 + iota) * (2 * b) + (b - 1)])
            lvl_v[pl.ds(LVL_OFF[b] + j0, LANES)] = ent

    def chunk_body(z_v, out_v):
        @plsc.parallel_loop(0, GROUPS, unroll=4)
        def _(g):
            rows = g * LANES + iota
            r0 = g * LANES
            u = z_v[0, pl.ds(r0, LANES)]
            pos_r = jnp.zeros((LANES,), jnp.int32)
            for b in (8, 4, 2, 1):
                val = cvec.at[pos_r + (b - 1)].get(mode="promise_in_bounds")
                pos_r = jnp.where(val <= u, pos_r + b, pos_r)
            pos = pos_r * 64
            for b in LVL_BITS:
                j = pos // (2 * b)
                val = plsc.load_gather(lvl_v, [j + LVL_OFF[b]])
                pos = jnp.where(val <= u, pos + b, pos)
            idx = jnp.minimum(pos, K - 1)
            xs = [z_v[1 + j, pl.ds(r0, LANES)] for j in range(D)]
            base_m = idx * MPAD
            base_d = idx * DPAD
            accs = [plsc.load_gather(means_v, [base_m + i]) for i in range(D)]
            for i in range(D):
                acc = accs[i]
                for j in range(D):
                    m = plsc.load_gather(devs_v, [base_d + (i * D + j)])
                    acc = acc + m * xs[j]
                accs[i] = acc
            for i in range(D):
                plsc.store_scatter(
                    out_v, [rows, jnp.full((LANES,), i, jnp.int32)], accs[i])

    pltpu.emit_pipeline(
        chunk_body,
        grid=(NUM_CHUNKS,),
        in_specs=[pl.BlockSpec((ZW, CHUNK_ROWS), lambda i: (0, i))],
        out_specs=[pl.BlockSpec((CHUNK_ROWS, D), lambda i: (i, 0))],
        core_axis_name=("c", "s"),
        dimension_semantics=(pltpu.PARALLEL,),
    )(z_hbm, out_hbm)


@jax.jit
def kernel(z, means, devs, mix_partition):
    mesh = plsc.VectorSubcoreMesh(core_axis_name="c", subcore_axis_name="s")
    cp = pltpu.CompilerParams()
    if "needs_layout_passes" in pltpu.CompilerParams.__dataclass_fields__:
        cp = dataclasses.replace(cp, needs_layout_passes=False)
    run = pl.kernel(
        _sc_body,
        out_type=jax.ShapeDtypeStruct((NP2, D), jnp.float32),
        mesh=mesh,
        scratch_types=[
            pltpu.VMEM((K,), jnp.float32),
            pltpu.VMEM((K * MPAD,), jnp.float32),
            pltpu.VMEM((K * DPAD,), jnp.float32),
            pltpu.VMEM((LVL_WORDS,), jnp.float32),
            pltpu.SemaphoreType.DMA,
        ],
        compiler_params=cp,
    )
    means_p = jnp.pad(means.reshape(K, D), ((0, 0), (0, MPAD - D))).reshape(K * MPAD)
    devs_p = jnp.pad(devs.reshape(K, D * D), ((0, 0), (0, DPAD - D * D))).reshape(K * DPAD)
    zt = jnp.pad(z.T, ((0, 0), (0, NP2 - N)))
    return run(zt, means_p, devs_p, mix_partition)[:N]


# speedup vs baseline: 7.0580x; 2.8734x over previous
import dataclasses

import jax
import jax.numpy as jnp
from jax import lax
from jax.experimental import pallas as pl
from jax.experimental.pallas import tpu as pltpu
from jax.experimental.pallas import tpu_sc as plsc

N = 1000000
NP2 = 1048576
D = 8
K = 1024
LANES = 16

CHUNK_ROWS = 128
GROUPS = CHUNK_ROWS // LANES
NUM_CHUNKS = NP2 // CHUNK_ROWS
ZW = D + 1
MPAD = D + 1
DPAD = D * D + 1

LVL_BITS = (32, 16, 8, 4, 2, 1)
LVL_OFF = {}
_off = 0
for _b in LVL_BITS:
    LVL_OFF[_b] = _off
    _off += K // (2 * _b)
LVL_WORDS = _off


def _sc_body(z_hbm, means_hbm, devs_hbm, part_hbm, out_hbm,
             part_v, means_v, devs_v, lvl_v, sem):
    c1 = pltpu.async_copy(part_hbm, part_v, sem)
    c2 = pltpu.async_copy(means_hbm, means_v, sem)
    c3 = pltpu.async_copy(devs_hbm, devs_v, sem)
    c1.wait()
    c2.wait()
    c3.wait()

    iota = lax.iota(jnp.int32, LANES)
    iota_z = iota * ZW
    iota_y = iota * D

    cvec = plsc.load_gather(part_v, [iota * 64 + 63])

    for b in LVL_BITS:
        n_ent = K // (2 * b)
        for j0 in range(0, n_ent, LANES):
            ent = plsc.load_gather(part_v, [(j0 + iota) * (2 * b) + (b - 1)])
            lvl_v[pl.ds(LVL_OFF[b] + j0, LANES)] = ent

    def chunk_body(z_v, out_v):
        @plsc.parallel_loop(0, GROUPS, unroll=4)
        def _(g):
            rows = g * LANES + iota
            r0 = g * LANES
            u = z_v[0, pl.ds(r0, LANES)]
            pos_r = jnp.zeros((LANES,), jnp.int32)
            for b in (8, 4, 2, 1):
                val = cvec.at[pos_r + (b - 1)].get(mode="promise_in_bounds")
                pos_r = jnp.where(val <= u, pos_r + b, pos_r)
            pos = pos_r * 64
            for b in LVL_BITS:
                j = pos // (2 * b)
                val = plsc.load_gather(lvl_v, [j + LVL_OFF[b]])
                pos = jnp.where(val <= u, pos + b, pos)
            idx = jnp.minimum(pos, K - 1)
            xs = [z_v[1 + j, pl.ds(r0, LANES)] for j in range(D)]
            base_m = idx * MPAD
            base_d = idx * DPAD
            accs = [plsc.load_gather(means_v, [base_m + i]) for i in range(D)]
            for i in range(D):
                acc = accs[i]
                for j in range(D):
                    m = plsc.load_gather(devs_v, [base_d + (i * D + j)])
                    acc = acc + m * xs[j]
                accs[i] = acc
            for i in range(D):
                out_v[i, pl.ds(r0, LANES)] = accs[i]

    pltpu.emit_pipeline(
        chunk_body,
        grid=(NUM_CHUNKS,),
        in_specs=[pl.BlockSpec((ZW, CHUNK_ROWS), lambda i: (0, i))],
        out_specs=[pl.BlockSpec((D, CHUNK_ROWS), lambda i: (0, i))],
        core_axis_name=("c", "s"),
        dimension_semantics=(pltpu.PARALLEL,),
    )(z_hbm, out_hbm)


@jax.jit
def kernel(z, means, devs, mix_partition):
    mesh = plsc.VectorSubcoreMesh(core_axis_name="c", subcore_axis_name="s")
    cp = pltpu.CompilerParams()
    if "needs_layout_passes" in pltpu.CompilerParams.__dataclass_fields__:
        cp = dataclasses.replace(cp, needs_layout_passes=False)
    run = pl.kernel(
        _sc_body,
        out_type=jax.ShapeDtypeStruct((D, NP2), jnp.float32),
        mesh=mesh,
        scratch_types=[
            pltpu.VMEM((K,), jnp.float32),
            pltpu.VMEM((K * MPAD,), jnp.float32),
            pltpu.VMEM((K * DPAD,), jnp.float32),
            pltpu.VMEM((LVL_WORDS,), jnp.float32),
            pltpu.SemaphoreType.DMA,
        ],
        compiler_params=cp,
    )
    means_p = jnp.pad(means.reshape(K, D), ((0, 0), (0, MPAD - D))).reshape(K * MPAD)
    devs_p = jnp.pad(devs.reshape(K, D * D), ((0, 0), (0, DPAD - D * D))).reshape(K * DPAD)
    zt = jnp.pad(z.T, ((0, 0), (0, NP2 - N)))
    return run(zt, means_p, devs_p, mix_partition)[:, :N].T


# speedup vs baseline: 7.3811x; 1.0458x over previous
import dataclasses

import jax
import jax.numpy as jnp
from jax import lax
from jax.experimental import pallas as pl
from jax.experimental.pallas import tpu as pltpu
from jax.experimental.pallas import tpu_sc as plsc

N = 1000000
NP2 = 1048576
D = 8
K = 1024
LANES = 16

CHUNK_ROWS = 512
GROUPS = CHUNK_ROWS // LANES
NUM_CHUNKS = NP2 // CHUNK_ROWS
ZW = D + 1
MPAD = D + 1
DPAD = D * D + 1

LVL_BITS = (32, 16, 8, 4, 2, 1)
LVL_OFF = {}
_off = 0
for _b in LVL_BITS:
    LVL_OFF[_b] = _off
    _off += K // (2 * _b)
LVL_WORDS = _off


def _sc_body(z_hbm, means_hbm, devs_hbm, part_hbm, out_hbm,
             part_v, means_v, devs_v, lvl_v, sem):
    c1 = pltpu.async_copy(part_hbm, part_v, sem)
    c2 = pltpu.async_copy(means_hbm, means_v, sem)
    c3 = pltpu.async_copy(devs_hbm, devs_v, sem)
    c1.wait()
    c2.wait()
    c3.wait()

    iota = lax.iota(jnp.int32, LANES)
    iota_z = iota * ZW
    iota_y = iota * D

    cvec = plsc.load_gather(part_v, [iota * 64 + 63])

    for b in LVL_BITS:
        n_ent = K // (2 * b)
        for j0 in range(0, n_ent, LANES):
            ent = plsc.load_gather(part_v, [(j0 + iota) * (2 * b) + (b - 1)])
            lvl_v[pl.ds(LVL_OFF[b] + j0, LANES)] = ent

    def chunk_body(z_v, out_v):
        @plsc.parallel_loop(0, GROUPS, unroll=4)
        def _(g):
            rows = g * LANES + iota
            r0 = g * LANES
            u = z_v[0, pl.ds(r0, LANES)]
            pos_r = jnp.zeros((LANES,), jnp.int32)
            for b in (8, 4, 2, 1):
                val = cvec.at[pos_r + (b - 1)].get(mode="promise_in_bounds")
                pos_r = jnp.where(val <= u, pos_r + b, pos_r)
            pos = pos_r * 64
            for b in LVL_BITS:
                j = pos // (2 * b)
                val = plsc.load_gather(lvl_v, [j + LVL_OFF[b]])
                pos = jnp.where(val <= u, pos + b, pos)
            idx = jnp.minimum(pos, K - 1)
            xs = [z_v[1 + j, pl.ds(r0, LANES)] for j in range(D)]
            base_m = idx * MPAD
            base_d = idx * DPAD
            accs = [plsc.load_gather(means_v, [base_m + i]) for i in range(D)]
            for i in range(D):
                acc = accs[i]
                for j in range(D):
                    m = plsc.load_gather(devs_v, [base_d + (i * D + j)])
                    acc = acc + m * xs[j]
                accs[i] = acc
            for i in range(D):
                out_v[i, pl.ds(r0, LANES)] = accs[i]

    pltpu.emit_pipeline(
        chunk_body,
        grid=(NUM_CHUNKS,),
        in_specs=[pl.BlockSpec((ZW, CHUNK_ROWS), lambda i: (0, i))],
        out_specs=[pl.BlockSpec((D, CHUNK_ROWS), lambda i: (0, i))],
        core_axis_name=("c", "s"),
        dimension_semantics=(pltpu.PARALLEL,),
    )(z_hbm, out_hbm)


@jax.jit
def kernel(z, means, devs, mix_partition):
    mesh = plsc.VectorSubcoreMesh(core_axis_name="c", subcore_axis_name="s")
    cp = pltpu.CompilerParams()
    if "needs_layout_passes" in pltpu.CompilerParams.__dataclass_fields__:
        cp = dataclasses.replace(cp, needs_layout_passes=False)
    run = pl.kernel(
        _sc_body,
        out_type=jax.ShapeDtypeStruct((D, NP2), jnp.float32),
        mesh=mesh,
        scratch_types=[
            pltpu.VMEM((K,), jnp.float32),
            pltpu.VMEM((K * MPAD,), jnp.float32),
            pltpu.VMEM((K * DPAD,), jnp.float32),
            pltpu.VMEM((LVL_WORDS,), jnp.float32),
            pltpu.SemaphoreType.DMA,
        ],
        compiler_params=cp,
    )
    means_p = jnp.pad(means.reshape(K, D), ((0, 0), (0, MPAD - D))).reshape(K * MPAD)
    devs_p = jnp.pad(devs.reshape(K, D * D), ((0, 0), (0, DPAD - D * D))).reshape(K * DPAD)
    zt = jnp.pad(z.T, ((0, 0), (0, NP2 - N)))
    return run(zt, means_p, devs_p, mix_partition)[:, :N].T
